# baseline (speedup 1.0000x reference)
import functools

import jax
import jax.numpy as jnp
from jax import lax
from jax.experimental import pallas as pl
from jax.experimental.pallas import tpu as pltpu
from jax.experimental.pallas import tpu_sc as plsc

T, H, E, K, I, C = 2048, 2048, 8, 2, 5504, 512
TRASH = E * C
BUF_ROWS = E * C + C
TT = 256
IT = 128
NC, NS = 2, 16
NW = NC * NS
TOK_W = T // NW
CHUNK = 16


def _plan_body(x_ref, wr_ref, logits_ref, s0_ref, s1_ref, g0_ref, g1_ref,
               w0_ref, w1_ref, base_ref):
    step = pl.program_id(0)

    @pl.when(step == 0)
    def _():
        base_ref[...] = jnp.zeros_like(base_ref)

    x = x_ref[...]
    logits = jnp.dot(x, wr_ref[...], preferred_element_type=jnp.float32)
    logits_ref[...] = logits

    m = jnp.max(logits, axis=-1, keepdims=True)
    ex = jnp.exp(logits - m)
    p = ex / jnp.sum(ex, axis=-1, keepdims=True)

    iota_e = lax.broadcasted_iota(jnp.int32, (TT, E), 1)
    v0 = jnp.max(p, axis=-1, keepdims=True)
    i0 = jnp.min(jnp.where(p == v0, iota_e, E), axis=-1, keepdims=True)
    pm = jnp.where(iota_e == i0, -1.0, p)
    v1 = jnp.max(pm, axis=-1, keepdims=True)
    i1 = jnp.min(jnp.where(pm == v1, iota_e, E), axis=-1, keepdims=True)
    s = v0 + v1
    w0 = v0 / s
    w1 = v1 / s

    oh0 = (iota_e == i0).astype(jnp.float32)
    oh1 = (iota_e == i1).astype(jnp.float32)
    ohs = oh0 + oh1
    r = lax.broadcasted_iota(jnp.int32, (TT, TT), 0)
    c = lax.broadcasted_iota(jnp.int32, (TT, TT), 1)
    ltri = (r > c).astype(jnp.float32)
    cnt = base_ref[...] + jnp.dot(ltri, ohs, preferred_element_type=jnp.float32)
    pos0 = jnp.sum(cnt * oh0, axis=-1, keepdims=True)
    pos1 = jnp.sum(cnt * oh1, axis=-1, keepdims=True)
    keep0 = pos0 < C
    keep1 = pos1 < C
    slot0 = jnp.where(keep0, i0 * C + pos0.astype(jnp.int32), TRASH)
    slot1 = jnp.where(keep1, i1 * C + pos1.astype(jnp.int32), TRASH)

    s0_ref[...] = slot0
    s1_ref[...] = slot1
    g0_ref[...] = jnp.minimum(slot0, TRASH - 1)
    g1_ref[...] = jnp.minimum(slot1, TRASH - 1)
    w0_ref[...] = jnp.where(keep0, w0, 0.0)
    w1_ref[...] = jnp.where(keep1, w1, 0.0)
    base_ref[...] = base_ref[...] + jnp.sum(ohs, axis=0, keepdims=True)


def _plan(x, wr):
    col_i = pl.BlockSpec((TT, 1), lambda i: (i, 0))
    return pl.pallas_call(
        _plan_body,
        grid=(T // TT,),
        in_specs=[pl.BlockSpec((TT, H), lambda i: (i, 0)),
                  pl.BlockSpec((H, E), lambda i: (0, 0))],
        out_specs=[pl.BlockSpec((TT, E), lambda i: (i, 0)),
                   col_i, col_i, col_i, col_i, col_i, col_i],
        out_shape=[jax.ShapeDtypeStruct((T, E), jnp.float32),
                   jax.ShapeDtypeStruct((T, 1), jnp.int32),
                   jax.ShapeDtypeStruct((T, 1), jnp.int32),
                   jax.ShapeDtypeStruct((T, 1), jnp.int32),
                   jax.ShapeDtypeStruct((T, 1), jnp.int32),
                   jax.ShapeDtypeStruct((T, 1), jnp.float32),
                   jax.ShapeDtypeStruct((T, 1), jnp.float32)],
        scratch_shapes=[pltpu.VMEM((1, E), jnp.float32)],
    )(x, wr)


def _sc_mesh():
    return plsc.VectorSubcoreMesh(core_axis_name="c", subcore_axis_name="s",
                                  num_cores=NC)


def _dispatch_body(x_hbm, s0_hbm, s1_hbm, buf_hbm,
                   xr_a, i0_a, i1_a, xr_b, i0_b, i1_b,
                   ssem_a, ssem_b, w0_a, w1_a, w0_b, w1_b):
    wid = lax.axis_index("s") * NC + lax.axis_index("c")
    base = wid * TOK_W
    nch = TOK_W // CHUNK
    slots = ((xr_a, i0_a, i1_a, ssem_a, w0_a, w1_a),
             (xr_b, i0_b, i1_b, ssem_b, w0_b, w1_b))
    stage_cp = [None, None]
    scat_cp = [None, None]

    def stage(c, sl):
        xr, i0, i1, ssem, _, _ = slots[sl]
        tb = base + c * CHUNK
        pltpu.sync_copy(s0_hbm.at[pl.ds(tb, CHUNK)], i0)
        pltpu.sync_copy(s1_hbm.at[pl.ds(tb, CHUNK)], i1)
        stage_cp[sl] = pltpu.async_copy(x_hbm.at[pl.ds(tb, CHUNK)], xr, ssem)

    stage(0, 0)
    for c in range(nch):
        sl = c % 2
        xr, i0, i1, _, ws0, ws1 = slots[sl]
        stage_cp[sl].wait()
        scat_cp[sl] = (pltpu.async_copy(xr, buf_hbm.at[i0], ws0),
                       pltpu.async_copy(xr, buf_hbm.at[i1], ws1))
        if c + 1 < nch:
            nsl = (c + 1) % 2
            if scat_cp[nsl] is not None:
                scat_cp[nsl][0].wait()
                scat_cp[nsl][1].wait()
            stage(c + 1, nsl)
    last = (nch - 1) % 2
    for sl in (1 - last, last):
        if scat_cp[sl] is not None:
            scat_cp[sl][0].wait()
            scat_cp[sl][1].wait()


def _dispatch(x, s0, s1):
    return pl.kernel(
        _dispatch_body,
        out_type=jax.ShapeDtypeStruct((BUF_ROWS, H), jnp.float32),
        mesh=_sc_mesh(),
        scratch_types=[pltpu.VMEM((CHUNK, H), jnp.float32),
                       pltpu.VMEM((CHUNK,), jnp.int32),
                       pltpu.VMEM((CHUNK,), jnp.int32),
                       pltpu.VMEM((CHUNK, H), jnp.float32),
                       pltpu.VMEM((CHUNK,), jnp.int32),
                       pltpu.VMEM((CHUNK,), jnp.int32),
                       pltpu.SemaphoreType.DMA, pltpu.SemaphoreType.DMA,
                       pltpu.SemaphoreType.DMA, pltpu.SemaphoreType.DMA,
                       pltpu.SemaphoreType.DMA, pltpu.SemaphoreType.DMA],
    )(x, s0, s1)


NPAIR = 21


def _mlp_body(bf16, buf_ref, wgp_ref, wup_ref, wdp_ref, wgs_ref, wus_ref,
              wds_ref, y_ref, xbf_ref):
    it = pl.program_id(1)
    cdt = jnp.bfloat16 if bf16 else jnp.float32

    @pl.when(it == 0)
    def _():
        y_ref[...] = jnp.zeros_like(y_ref)
        if bf16:
            xbf_ref[...] = buf_ref[...].astype(jnp.bfloat16)

    x = xbf_ref[...] if bf16 else buf_ref[...]

    def glu(wg2, wu2, wd2):
        g = jnp.dot(x, wg2.astype(cdt), preferred_element_type=jnp.float32)
        u = jnp.dot(x, wu2.astype(cdt), preferred_element_type=jnp.float32)
        h = (g * lax.logistic(g) * u).astype(cdt)
        y_ref[...] += jnp.dot(h, wd2.astype(cdt),
                              preferred_element_type=jnp.float32)

    @pl.when(it < NPAIR)
    def _():
        glu(wgp_ref[0], wup_ref[0], wdp_ref[0])

    @pl.when(it == NPAIR)
    def _():
        glu(wgs_ref[0], wus_ref[0], wds_ref[0])


def _mlp(buf, wg, wu, wd, bf16):
    nt = I // IT
    return pl.pallas_call(
        functools.partial(_mlp_body, bf16),
        grid=(E, NPAIR + 1),
        in_specs=[
            pl.BlockSpec((C, H), lambda e, i: (e, 0)),
            pl.BlockSpec((1, H, 2 * IT),
                         lambda e, i: (e, 0, jnp.minimum(i, NPAIR - 1))),
            pl.BlockSpec((1, H, 2 * IT),
                         lambda e, i: (e, 0, jnp.minimum(i, NPAIR - 1))),
            pl.BlockSpec((1, 2 * IT, H),
                         lambda e, i: (e, jnp.minimum(i, NPAIR - 1), 0)),
            pl.BlockSpec((1, H, IT), lambda e, i: (e, 0, nt - 1)),
            pl.BlockSpec((1, H, IT), lambda e, i: (e, 0, nt - 1)),
            pl.BlockSpec((1, IT, H), lambda e, i: (e, nt - 1, 0)),
        ],
        out_specs=pl.BlockSpec((C, H), lambda e, i: (e, 0)),
        out_shape=jax.ShapeDtypeStruct((E * C, H), jnp.float32),
        scratch_shapes=[pltpu.VMEM((C, H), jnp.bfloat16)],
    )(buf, wg, wu, wd, wg, wu, wd)


CHG = 8


def _gather_body(y_hbm, g0_hbm, g1_hbm, y0_hbm, y1_hbm,
                 r0_a, r1_a, i0_a, i1_a, r0_b, r1_b, i0_b, i1_b,
                 g0s_a, g1s_a, g0s_b, g1s_b, w0s_a, w1s_a, w0s_b, w1s_b):
    wid = lax.axis_index("s") * NC + lax.axis_index("c")
    base = wid * TOK_W
    nch = TOK_W // CHG
    slots = ((r0_a, r1_a, i0_a, i1_a, (g0s_a, g1s_a), (w0s_a, w1s_a)),
             (r0_b, r1_b, i0_b, i1_b, (g0s_b, g1s_b), (w0s_b, w1s_b)))
    gin = [None, None]
    wout = [None, None]

    def issue_gather(c, sl):
        r0, r1, i0, i1, gs, _ = slots[sl]
        tb = base + c * CHG
        pltpu.sync_copy(g0_hbm.at[pl.ds(tb, CHG)], i0)
        pltpu.sync_copy(g1_hbm.at[pl.ds(tb, CHG)], i1)
        gin[sl] = (pltpu.async_copy(y_hbm.at[i0], r0, gs[0]),
                   pltpu.async_copy(y_hbm.at[i1], r1, gs[1]))

    issue_gather(0, 0)
    for c in range(nch):
        sl = c % 2
        if c + 1 < nch:
            nsl = (c + 1) % 2
            if wout[nsl] is not None:
                wout[nsl][0].wait()
                wout[nsl][1].wait()
            issue_gather(c + 1, nsl)
        r0, r1, _, _, _, ws = slots[sl]
        gin[sl][0].wait()
        gin[sl][1].wait()
        tb = base + c * CHG
        wout[sl] = (pltpu.async_copy(r0, y0_hbm.at[pl.ds(tb, CHG)], ws[0]),
                    pltpu.async_copy(r1, y1_hbm.at[pl.ds(tb, CHG)], ws[1]))
    last = (nch - 1) % 2
    for sl in (1 - last, last):
        if wout[sl] is not None:
            wout[sl][0].wait()
            wout[sl][1].wait()


def _gather(y, g0, g1):
    return pl.kernel(
        _gather_body,
        out_type=[jax.ShapeDtypeStruct((T, H), jnp.float32),
                  jax.ShapeDtypeStruct((T, H), jnp.float32)],
        mesh=_sc_mesh(),
        scratch_types=[pltpu.VMEM((CHG, H), jnp.float32),
                       pltpu.VMEM((CHG, H), jnp.float32),
                       pltpu.VMEM((CHG,), jnp.int32),
                       pltpu.VMEM((CHG,), jnp.int32),
                       pltpu.VMEM((CHG, H), jnp.float32),
                       pltpu.VMEM((CHG, H), jnp.float32),
                       pltpu.VMEM((CHG,), jnp.int32),
                       pltpu.VMEM((CHG,), jnp.int32)]
                     + [pltpu.SemaphoreType.DMA] * 8,
    )(y, g0, g1)


def _combine_body(y0_ref, y1_ref, w0_ref, w1_ref, out_ref):
    w0 = w0_ref[...]
    w1 = w1_ref[...]
    a = jnp.where(w0 > 0, y0_ref[...] * w0, 0.0)
    b = jnp.where(w1 > 0, y1_ref[...] * w1, 0.0)
    out_ref[...] = a + b


def _combine(y0, y1, w0, w1):
    col_i = pl.BlockSpec((TT, 1), lambda i: (i, 0))
    return pl.pallas_call(
        _combine_body,
        grid=(T // TT,),
        in_specs=[pl.BlockSpec((TT, H), lambda i: (i, 0)),
                  pl.BlockSpec((TT, H), lambda i: (i, 0)),
                  col_i, col_i],
        out_specs=pl.BlockSpec((TT, H), lambda i: (i, 0)),
        out_shape=jax.ShapeDtypeStruct((T, H), jnp.float32),
    )(y0, y1, w0, w1)


def _layer(x, wr, wg, wu, wd, mlp_bf16):
    logits, s0, s1, g0, g1, w0, w1 = _plan(x, wr)
    buf = _dispatch(x, s0.reshape(T), s1.reshape(T))
    y = _mlp(buf, wg, wu, wd, mlp_bf16)
    y0, y1 = _gather(y, g0.reshape(T), g1.reshape(T))
    return _combine(y0, y1, w0, w1), logits


def kernel(hidden_states, Wr0, Wg0, Wu0, Wd0, Wr1, Wg1, Wu1, Wd1):
    x = hidden_states.reshape(T, H)
    x, rl0 = _layer(x, Wr0, Wg0, Wu0, Wd0, False)
    x, rl1 = _layer(x, Wr1, Wg1, Wu1, Wd1, True)
    op = x.reshape(hidden_states.shape)
    return op, jnp.concatenate([rl0, rl1], axis=0)


# speedup vs baseline: 1.0562x; 1.0562x over previous
import functools

import jax
import jax.numpy as jnp
from jax import lax
from jax.experimental import pallas as pl
from jax.experimental.pallas import tpu as pltpu
from jax.experimental.pallas import tpu_sc as plsc

T, H, E, K, I, C = 2048, 2048, 8, 2, 5504, 512
TRASH = E * C
BUF_ROWS = E * C + C
TT = 256
IT = 128
NC, NS = 2, 16
NW = NC * NS
TOK_W = T // NW
CHUNK = 16


def _plan_body(x_ref, wr_ref, logits_ref, s0_ref, s1_ref, g0_ref, g1_ref,
               w0_ref, w1_ref, base_ref):
    step = pl.program_id(0)

    @pl.when(step == 0)
    def _():
        base_ref[...] = jnp.zeros_like(base_ref)

    x = x_ref[...]
    logits = jnp.dot(x, wr_ref[...], preferred_element_type=jnp.float32)
    logits_ref[...] = logits

    m = jnp.max(logits, axis=-1, keepdims=True)
    ex = jnp.exp(logits - m)
    p = ex / jnp.sum(ex, axis=-1, keepdims=True)

    iota_e = lax.broadcasted_iota(jnp.int32, (TT, E), 1)
    v0 = jnp.max(p, axis=-1, keepdims=True)
    i0 = jnp.min(jnp.where(p == v0, iota_e, E), axis=-1, keepdims=True)
    pm = jnp.where(iota_e == i0, -1.0, p)
    v1 = jnp.max(pm, axis=-1, keepdims=True)
    i1 = jnp.min(jnp.where(pm == v1, iota_e, E), axis=-1, keepdims=True)
    s = v0 + v1
    w0 = v0 / s
    w1 = v1 / s

    oh0 = (iota_e == i0).astype(jnp.float32)
    oh1 = (iota_e == i1).astype(jnp.float32)
    ohs = oh0 + oh1
    r = lax.broadcasted_iota(jnp.int32, (TT, TT), 0)
    c = lax.broadcasted_iota(jnp.int32, (TT, TT), 1)
    ltri = (r > c).astype(jnp.float32)
    cnt = base_ref[...] + jnp.dot(ltri, ohs, preferred_element_type=jnp.float32)
    pos0 = jnp.sum(cnt * oh0, axis=-1, keepdims=True)
    pos1 = jnp.sum(cnt * oh1, axis=-1, keepdims=True)
    keep0 = pos0 < C
    keep1 = pos1 < C
    slot0 = jnp.where(keep0, i0 * C + pos0.astype(jnp.int32), TRASH)
    slot1 = jnp.where(keep1, i1 * C + pos1.astype(jnp.int32), TRASH)

    s0_ref[...] = slot0
    s1_ref[...] = slot1
    g0_ref[...] = jnp.minimum(slot0, TRASH - 1)
    g1_ref[...] = jnp.minimum(slot1, TRASH - 1)
    w0_ref[...] = jnp.where(keep0, w0, 0.0)
    w1_ref[...] = jnp.where(keep1, w1, 0.0)
    base_ref[...] = base_ref[...] + jnp.sum(ohs, axis=0, keepdims=True)


def _plan(x, wr):
    col_i = pl.BlockSpec((TT, 1), lambda i: (i, 0))
    return pl.pallas_call(
        _plan_body,
        grid=(T // TT,),
        in_specs=[pl.BlockSpec((TT, H), lambda i: (i, 0)),
                  pl.BlockSpec((H, E), lambda i: (0, 0))],
        out_specs=[pl.BlockSpec((TT, E), lambda i: (i, 0)),
                   col_i, col_i, col_i, col_i, col_i, col_i],
        out_shape=[jax.ShapeDtypeStruct((T, E), jnp.float32),
                   jax.ShapeDtypeStruct((T, 1), jnp.int32),
                   jax.ShapeDtypeStruct((T, 1), jnp.int32),
                   jax.ShapeDtypeStruct((T, 1), jnp.int32),
                   jax.ShapeDtypeStruct((T, 1), jnp.int32),
                   jax.ShapeDtypeStruct((T, 1), jnp.float32),
                   jax.ShapeDtypeStruct((T, 1), jnp.float32)],
        scratch_shapes=[pltpu.VMEM((1, E), jnp.float32)],
    )(x, wr)


def _sc_mesh():
    return plsc.VectorSubcoreMesh(core_axis_name="c", subcore_axis_name="s",
                                  num_cores=NC)


def _dispatch_body(x_hbm, s0_hbm, s1_hbm, buf_hbm,
                   xr_a, i0_a, i1_a, xr_b, i0_b, i1_b,
                   ssem_a, ssem_b, w0_a, w1_a, w0_b, w1_b):
    wid = lax.axis_index("s") * NC + lax.axis_index("c")
    base = wid * TOK_W
    nch = TOK_W // CHUNK
    slots = ((xr_a, i0_a, i1_a, ssem_a, w0_a, w1_a),
             (xr_b, i0_b, i1_b, ssem_b, w0_b, w1_b))
    stage_cp = [None, None]
    scat_cp = [None, None]

    def stage(c, sl):
        xr, i0, i1, ssem, _, _ = slots[sl]
        tb = base + c * CHUNK
        pltpu.sync_copy(s0_hbm.at[pl.ds(tb, CHUNK)], i0)
        pltpu.sync_copy(s1_hbm.at[pl.ds(tb, CHUNK)], i1)
        stage_cp[sl] = pltpu.async_copy(x_hbm.at[pl.ds(tb, CHUNK)], xr, ssem)

    stage(0, 0)
    for c in range(nch):
        sl = c % 2
        xr, i0, i1, _, ws0, ws1 = slots[sl]
        stage_cp[sl].wait()
        scat_cp[sl] = (pltpu.async_copy(xr, buf_hbm.at[i0], ws0),
                       pltpu.async_copy(xr, buf_hbm.at[i1], ws1))
        if c + 1 < nch:
            nsl = (c + 1) % 2
            if scat_cp[nsl] is not None:
                scat_cp[nsl][0].wait()
                scat_cp[nsl][1].wait()
            stage(c + 1, nsl)
    last = (nch - 1) % 2
    for sl in (1 - last, last):
        if scat_cp[sl] is not None:
            scat_cp[sl][0].wait()
            scat_cp[sl][1].wait()


def _dispatch(x, s0, s1):
    return pl.kernel(
        _dispatch_body,
        out_type=jax.ShapeDtypeStruct((BUF_ROWS, H), jnp.float32),
        mesh=_sc_mesh(),
        scratch_types=[pltpu.VMEM((CHUNK, H), jnp.float32),
                       pltpu.VMEM((CHUNK,), jnp.int32),
                       pltpu.VMEM((CHUNK,), jnp.int32),
                       pltpu.VMEM((CHUNK, H), jnp.float32),
                       pltpu.VMEM((CHUNK,), jnp.int32),
                       pltpu.VMEM((CHUNK,), jnp.int32),
                       pltpu.SemaphoreType.DMA, pltpu.SemaphoreType.DMA,
                       pltpu.SemaphoreType.DMA, pltpu.SemaphoreType.DMA,
                       pltpu.SemaphoreType.DMA, pltpu.SemaphoreType.DMA],
    )(x, s0, s1)


NQUAD = 10


def _mlp_body(bf16, buf_ref, wgq_ref, wuq_ref, wdq_ref, wgs_ref, wus_ref,
              wds_ref, y_ref, xbf_ref):
    it = pl.program_id(1)
    cdt = jnp.bfloat16 if bf16 else jnp.float32

    @pl.when(it == 0)
    def _():
        y_ref[...] = jnp.zeros_like(y_ref)
        if bf16:
            xbf_ref[...] = buf_ref[...].astype(jnp.bfloat16)

    x = xbf_ref[...] if bf16 else buf_ref[...]

    def glu(wg2, wu2, wd2):
        g = jnp.dot(x, wg2.astype(cdt), preferred_element_type=jnp.float32)
        u = jnp.dot(x, wu2.astype(cdt), preferred_element_type=jnp.float32)
        h = (g * lax.logistic(g) * u).astype(cdt)
        y_ref[...] += jnp.dot(h, wd2.astype(cdt),
                              preferred_element_type=jnp.float32)

    @pl.when(it < NQUAD)
    def _():
        glu(wgq_ref[0], wuq_ref[0], wdq_ref[0])

    @pl.when(it >= NQUAD)
    def _():
        glu(wgs_ref[0], wus_ref[0], wds_ref[0])


def _mlp(buf, wg, wu, wd, bf16):
    def smap_i(e, i):
        return (e, 0, jnp.minimum(jnp.maximum(i, NQUAD) + 4 * NQUAD - NQUAD,
                                  I // IT - 1))

    def smap_d(e, i):
        return (e, jnp.minimum(jnp.maximum(i, NQUAD) + 4 * NQUAD - NQUAD,
                               I // IT - 1), 0)

    return pl.pallas_call(
        functools.partial(_mlp_body, bf16),
        grid=(E, NQUAD + 3),
        in_specs=[
            pl.BlockSpec((C, H), lambda e, i: (e, 0)),
            pl.BlockSpec((1, H, 4 * IT),
                         lambda e, i: (e, 0, jnp.minimum(i, NQUAD - 1))),
            pl.BlockSpec((1, H, 4 * IT),
                         lambda e, i: (e, 0, jnp.minimum(i, NQUAD - 1))),
            pl.BlockSpec((1, 4 * IT, H),
                         lambda e, i: (e, jnp.minimum(i, NQUAD - 1), 0)),
            pl.BlockSpec((1, H, IT), smap_i),
            pl.BlockSpec((1, H, IT), smap_i),
            pl.BlockSpec((1, IT, H), smap_d),
        ],
        out_specs=pl.BlockSpec((C, H), lambda e, i: (e, 0)),
        out_shape=jax.ShapeDtypeStruct((E * C, H), jnp.float32),
        scratch_shapes=[pltpu.VMEM((C, H), jnp.bfloat16)],
    )(buf, wg, wu, wd, wg, wu, wd)


CHG = 8


def _gather_body(y_hbm, g0_hbm, g1_hbm, y0_hbm, y1_hbm,
                 r0_a, r1_a, i0_a, i1_a, r0_b, r1_b, i0_b, i1_b,
                 g0s_a, g1s_a, g0s_b, g1s_b, w0s_a, w1s_a, w0s_b, w1s_b):
    wid = lax.axis_index("s") * NC + lax.axis_index("c")
    base = wid * TOK_W
    nch = TOK_W // CHG
    slots = ((r0_a, r1_a, i0_a, i1_a, (g0s_a, g1s_a), (w0s_a, w1s_a)),
             (r0_b, r1_b, i0_b, i1_b, (g0s_b, g1s_b), (w0s_b, w1s_b)))
    gin = [None, None]
    wout = [None, None]

    def issue_gather(c, sl):
        r0, r1, i0, i1, gs, _ = slots[sl]
        tb = base + c * CHG
        pltpu.sync_copy(g0_hbm.at[pl.ds(tb, CHG)], i0)
        pltpu.sync_copy(g1_hbm.at[pl.ds(tb, CHG)], i1)
        gin[sl] = (pltpu.async_copy(y_hbm.at[i0], r0, gs[0]),
                   pltpu.async_copy(y_hbm.at[i1], r1, gs[1]))

    issue_gather(0, 0)
    for c in range(nch):
        sl = c % 2
        if c + 1 < nch:
            nsl = (c + 1) % 2
            if wout[nsl] is not None:
                wout[nsl][0].wait()
                wout[nsl][1].wait()
            issue_gather(c + 1, nsl)
        r0, r1, _, _, _, ws = slots[sl]
        gin[sl][0].wait()
        gin[sl][1].wait()
        tb = base + c * CHG
        wout[sl] = (pltpu.async_copy(r0, y0_hbm.at[pl.ds(tb, CHG)], ws[0]),
                    pltpu.async_copy(r1, y1_hbm.at[pl.ds(tb, CHG)], ws[1]))
    last = (nch - 1) % 2
    for sl in (1 - last, last):
        if wout[sl] is not None:
            wout[sl][0].wait()
            wout[sl][1].wait()


def _gather(y, g0, g1):
    return pl.kernel(
        _gather_body,
        out_type=[jax.ShapeDtypeStruct((T, H), jnp.float32),
                  jax.ShapeDtypeStruct((T, H), jnp.float32)],
        mesh=_sc_mesh(),
        scratch_types=[pltpu.VMEM((CHG, H), jnp.float32),
                       pltpu.VMEM((CHG, H), jnp.float32),
                       pltpu.VMEM((CHG,), jnp.int32),
                       pltpu.VMEM((CHG,), jnp.int32),
                       pltpu.VMEM((CHG, H), jnp.float32),
                       pltpu.VMEM((CHG, H), jnp.float32),
                       pltpu.VMEM((CHG,), jnp.int32),
                       pltpu.VMEM((CHG,), jnp.int32)]
                     + [pltpu.SemaphoreType.DMA] * 8,
    )(y, g0, g1)


def _combine_body(y0_ref, y1_ref, w0_ref, w1_ref, out_ref):
    w0 = w0_ref[...]
    w1 = w1_ref[...]
    a = jnp.where(w0 > 0, y0_ref[...] * w0, 0.0)
    b = jnp.where(w1 > 0, y1_ref[...] * w1, 0.0)
    out_ref[...] = a + b


def _combine(y0, y1, w0, w1):
    col_i = pl.BlockSpec((TT, 1), lambda i: (i, 0))
    return pl.pallas_call(
        _combine_body,
        grid=(T // TT,),
        in_specs=[pl.BlockSpec((TT, H), lambda i: (i, 0)),
                  pl.BlockSpec((TT, H), lambda i: (i, 0)),
                  col_i, col_i],
        out_specs=pl.BlockSpec((TT, H), lambda i: (i, 0)),
        out_shape=jax.ShapeDtypeStruct((T, H), jnp.float32),
    )(y0, y1, w0, w1)


def _layer(x, wr, wg, wu, wd, mlp_bf16):
    logits, s0, s1, g0, g1, w0, w1 = _plan(x, wr)
    buf = _dispatch(x, s0.reshape(T), s1.reshape(T))
    y = _mlp(buf, wg, wu, wd, mlp_bf16)
    y0, y1 = _gather(y, g0.reshape(T), g1.reshape(T))
    return _combine(y0, y1, w0, w1), logits


def kernel(hidden_states, Wr0, Wg0, Wu0, Wd0, Wr1, Wg1, Wu1, Wd1):
    x = hidden_states.reshape(T, H)
    x, rl0 = _layer(x, Wr0, Wg0, Wu0, Wd0, False)
    x, rl1 = _layer(x, Wr1, Wg1, Wu1, Wd1, True)
    op = x.reshape(hidden_states.shape)
    return op, jnp.concatenate([rl0, rl1], axis=0)


# speedup vs baseline: 1.0678x; 1.0110x over previous
import functools

import jax
import jax.numpy as jnp
from jax import lax
from jax.experimental import pallas as pl
from jax.experimental.pallas import tpu as pltpu
from jax.experimental.pallas import tpu_sc as plsc

T, H, E, K, I, C = 2048, 2048, 8, 2, 5504, 512
TRASH = E * C
BUF_ROWS = E * C + C
TT = 256
IT = 128
NC, NS = 2, 16
NW = NC * NS
TOK_W = T // NW
CHUNK = 16


def _plan_math(x, wr_ref, logits_ref, s0_ref, s1_ref, g0_ref, g1_ref,
               w0_ref, w1_ref, base_ref):
    logits = jnp.dot(x, wr_ref[...], preferred_element_type=jnp.float32)
    logits_ref[...] = logits

    m = jnp.max(logits, axis=-1, keepdims=True)
    ex = jnp.exp(logits - m)
    p = ex / jnp.sum(ex, axis=-1, keepdims=True)

    iota_e = lax.broadcasted_iota(jnp.int32, (TT, E), 1)
    v0 = jnp.max(p, axis=-1, keepdims=True)
    i0 = jnp.min(jnp.where(p == v0, iota_e, E), axis=-1, keepdims=True)
    pm = jnp.where(iota_e == i0, -1.0, p)
    v1 = jnp.max(pm, axis=-1, keepdims=True)
    i1 = jnp.min(jnp.where(pm == v1, iota_e, E), axis=-1, keepdims=True)
    s = v0 + v1
    w0 = v0 / s
    w1 = v1 / s

    oh0 = (iota_e == i0).astype(jnp.float32)
    oh1 = (iota_e == i1).astype(jnp.float32)
    ohs = oh0 + oh1
    r = lax.broadcasted_iota(jnp.int32, (TT, TT), 0)
    c = lax.broadcasted_iota(jnp.int32, (TT, TT), 1)
    ltri = (r > c).astype(jnp.float32)
    cnt = base_ref[...] + jnp.dot(ltri, ohs, preferred_element_type=jnp.float32)
    pos0 = jnp.sum(cnt * oh0, axis=-1, keepdims=True)
    pos1 = jnp.sum(cnt * oh1, axis=-1, keepdims=True)
    keep0 = pos0 < C
    keep1 = pos1 < C
    slot0 = jnp.where(keep0, i0 * C + pos0.astype(jnp.int32), TRASH)
    slot1 = jnp.where(keep1, i1 * C + pos1.astype(jnp.int32), TRASH)

    s0_ref[...] = slot0
    s1_ref[...] = slot1
    g0_ref[...] = jnp.minimum(slot0, TRASH - 1)
    g1_ref[...] = jnp.minimum(slot1, TRASH - 1)
    w0_ref[...] = jnp.where(keep0, w0, 0.0)
    w1_ref[...] = jnp.where(keep1, w1, 0.0)
    base_ref[...] = base_ref[...] + jnp.sum(ohs, axis=0, keepdims=True)


def _plan_body(x_ref, wr_ref, logits_ref, s0_ref, s1_ref, g0_ref, g1_ref,
               w0_ref, w1_ref, base_ref):
    @pl.when(pl.program_id(0) == 0)
    def _():
        base_ref[...] = jnp.zeros_like(base_ref)

    _plan_math(x_ref[...], wr_ref, logits_ref, s0_ref, s1_ref, g0_ref,
               g1_ref, w0_ref, w1_ref, base_ref)


_PLAN_OUT_SHAPE = [jax.ShapeDtypeStruct((T, E), jnp.float32),
                   jax.ShapeDtypeStruct((T, 1), jnp.int32),
                   jax.ShapeDtypeStruct((T, 1), jnp.int32),
                   jax.ShapeDtypeStruct((T, 1), jnp.int32),
                   jax.ShapeDtypeStruct((T, 1), jnp.int32),
                   jax.ShapeDtypeStruct((T, 1), jnp.float32),
                   jax.ShapeDtypeStruct((T, 1), jnp.float32)]


def _plan(x, wr):
    col_i = pl.BlockSpec((TT, 1), lambda i: (i, 0))
    return pl.pallas_call(
        _plan_body,
        grid=(T // TT,),
        in_specs=[pl.BlockSpec((TT, H), lambda i: (i, 0)),
                  pl.BlockSpec((H, E), lambda i: (0, 0))],
        out_specs=[pl.BlockSpec((TT, E), lambda i: (i, 0)),
                   col_i, col_i, col_i, col_i, col_i, col_i],
        out_shape=list(_PLAN_OUT_SHAPE),
        scratch_shapes=[pltpu.VMEM((1, E), jnp.float32)],
    )(x, wr)


def _combine_plan_body(y0_ref, y1_ref, cw0_ref, cw1_ref, wr_ref, x_ref,
                       logits_ref, s0_ref, s1_ref, g0_ref, g1_ref,
                       w0_ref, w1_ref, base_ref):
    @pl.when(pl.program_id(0) == 0)
    def _():
        base_ref[...] = jnp.zeros_like(base_ref)

    cw0 = cw0_ref[...]
    cw1 = cw1_ref[...]
    x = (jnp.where(cw0 > 0, y0_ref[...] * cw0, 0.0)
         + jnp.where(cw1 > 0, y1_ref[...] * cw1, 0.0))
    x_ref[...] = x
    _plan_math(x, wr_ref, logits_ref, s0_ref, s1_ref, g0_ref, g1_ref,
               w0_ref, w1_ref, base_ref)


def _combine_plan(y0, y1, cw0, cw1, wr):
    col_i = pl.BlockSpec((TT, 1), lambda i: (i, 0))
    return pl.pallas_call(
        _combine_plan_body,
        grid=(T // TT,),
        in_specs=[pl.BlockSpec((TT, H), lambda i: (i, 0)),
                  pl.BlockSpec((TT, H), lambda i: (i, 0)),
                  col_i, col_i,
                  pl.BlockSpec((H, E), lambda i: (0, 0))],
        out_specs=[pl.BlockSpec((TT, H), lambda i: (i, 0)),
                   pl.BlockSpec((TT, E), lambda i: (i, 0)),
                   col_i, col_i, col_i, col_i, col_i, col_i],
        out_shape=[jax.ShapeDtypeStruct((T, H), jnp.float32)]
                  + list(_PLAN_OUT_SHAPE),
        scratch_shapes=[pltpu.VMEM((1, E), jnp.float32)],
    )(y0, y1, cw0, cw1, wr)


def _sc_mesh():
    return plsc.VectorSubcoreMesh(core_axis_name="c", subcore_axis_name="s",
                                  num_cores=NC)


def _dispatch_body(x_hbm, s0_hbm, s1_hbm, buf_hbm,
                   xr_a, i0_a, i1_a, xr_b, i0_b, i1_b,
                   ssem_a, ssem_b, w0_a, w1_a, w0_b, w1_b):
    wid = lax.axis_index("s") * NC + lax.axis_index("c")
    base = wid * TOK_W
    nch = TOK_W // CHUNK
    slots = ((xr_a, i0_a, i1_a, ssem_a, w0_a, w1_a),
             (xr_b, i0_b, i1_b, ssem_b, w0_b, w1_b))
    stage_cp = [None, None]
    scat_cp = [None, None]

    def stage(c, sl):
        xr, i0, i1, ssem, _, _ = slots[sl]
        tb = base + c * CHUNK
        pltpu.sync_copy(s0_hbm.at[pl.ds(tb, CHUNK)], i0)
        pltpu.sync_copy(s1_hbm.at[pl.ds(tb, CHUNK)], i1)
        stage_cp[sl] = pltpu.async_copy(x_hbm.at[pl.ds(tb, CHUNK)], xr, ssem)

    stage(0, 0)
    for c in range(nch):
        sl = c % 2
        xr, i0, i1, _, ws0, ws1 = slots[sl]
        stage_cp[sl].wait()
        scat_cp[sl] = (pltpu.async_copy(xr, buf_hbm.at[i0], ws0),
                       pltpu.async_copy(xr, buf_hbm.at[i1], ws1))
        if c + 1 < nch:
            nsl = (c + 1) % 2
            if scat_cp[nsl] is not None:
                scat_cp[nsl][0].wait()
                scat_cp[nsl][1].wait()
            stage(c + 1, nsl)
    last = (nch - 1) % 2
    for sl in (1 - last, last):
        if scat_cp[sl] is not None:
            scat_cp[sl][0].wait()
            scat_cp[sl][1].wait()


def _dispatch(x, s0, s1):
    return pl.kernel(
        _dispatch_body,
        out_type=jax.ShapeDtypeStruct((BUF_ROWS, H), jnp.float32),
        mesh=_sc_mesh(),
        scratch_types=[pltpu.VMEM((CHUNK, H), jnp.float32),
                       pltpu.VMEM((CHUNK,), jnp.int32),
                       pltpu.VMEM((CHUNK,), jnp.int32),
                       pltpu.VMEM((CHUNK, H), jnp.float32),
                       pltpu.VMEM((CHUNK,), jnp.int32),
                       pltpu.VMEM((CHUNK,), jnp.int32),
                       pltpu.SemaphoreType.DMA, pltpu.SemaphoreType.DMA,
                       pltpu.SemaphoreType.DMA, pltpu.SemaphoreType.DMA,
                       pltpu.SemaphoreType.DMA, pltpu.SemaphoreType.DMA],
    )(x, s0, s1)


NQUAD = 10


def _mlp_body(bf16, buf_ref, wgq_ref, wuq_ref, wdq_ref, wgs_ref, wus_ref,
              wds_ref, y_ref, xbf_ref):
    it = pl.program_id(1)
    cdt = jnp.bfloat16 if bf16 else jnp.float32

    @pl.when(it == 0)
    def _():
        y_ref[...] = jnp.zeros_like(y_ref)
        if bf16:
            xbf_ref[...] = buf_ref[...].astype(jnp.bfloat16)

    x = xbf_ref[...] if bf16 else buf_ref[...]

    def glu(wg2, wu2, wd2):
        g = jnp.dot(x, wg2.astype(cdt), preferred_element_type=jnp.float32)
        u = jnp.dot(x, wu2.astype(cdt), preferred_element_type=jnp.float32)
        h = (g * lax.logistic(g) * u).astype(cdt)
        y_ref[...] += jnp.dot(h, wd2.astype(cdt),
                              preferred_element_type=jnp.float32)

    @pl.when(it < NQUAD)
    def _():
        glu(wgq_ref[0], wuq_ref[0], wdq_ref[0])

    @pl.when(it >= NQUAD)
    def _():
        glu(wgs_ref[0], wus_ref[0], wds_ref[0])


def _mlp(buf, wg, wu, wd, bf16):
    def smap_i(e, i):
        return (e, 0, jnp.minimum(jnp.maximum(i, NQUAD) + 4 * NQUAD - NQUAD,
                                  I // IT - 1))

    def smap_d(e, i):
        return (e, jnp.minimum(jnp.maximum(i, NQUAD) + 4 * NQUAD - NQUAD,
                               I // IT - 1), 0)

    return pl.pallas_call(
        functools.partial(_mlp_body, bf16),
        grid=(E, NQUAD + 3),
        in_specs=[
            pl.BlockSpec((C, H), lambda e, i: (e, 0)),
            pl.BlockSpec((1, H, 4 * IT),
                         lambda e, i: (e, 0, jnp.minimum(i, NQUAD - 1))),
            pl.BlockSpec((1, H, 4 * IT),
                         lambda e, i: (e, 0, jnp.minimum(i, NQUAD - 1))),
            pl.BlockSpec((1, 4 * IT, H),
                         lambda e, i: (e, jnp.minimum(i, NQUAD - 1), 0)),
            pl.BlockSpec((1, H, IT), smap_i),
            pl.BlockSpec((1, H, IT), smap_i),
            pl.BlockSpec((1, IT, H), smap_d),
        ],
        out_specs=pl.BlockSpec((C, H), lambda e, i: (e, 0)),
        out_shape=jax.ShapeDtypeStruct((E * C, H), jnp.float32),
        scratch_shapes=[pltpu.VMEM((C, H), jnp.bfloat16)],
    )(buf, wg, wu, wd, wg, wu, wd)


CHG = 8


def _gather_body(y_hbm, g0_hbm, g1_hbm, y0_hbm, y1_hbm,
                 r0_a, r1_a, i0_a, i1_a, r0_b, r1_b, i0_b, i1_b,
                 g0s_a, g1s_a, g0s_b, g1s_b, w0s_a, w1s_a, w0s_b, w1s_b):
    wid = lax.axis_index("s") * NC + lax.axis_index("c")
    base = wid * TOK_W
    nch = TOK_W // CHG
    slots = ((r0_a, r1_a, i0_a, i1_a, (g0s_a, g1s_a), (w0s_a, w1s_a)),
             (r0_b, r1_b, i0_b, i1_b, (g0s_b, g1s_b), (w0s_b, w1s_b)))
    gin = [None, None]
    wout = [None, None]

    def issue_gather(c, sl):
        r0, r1, i0, i1, gs, _ = slots[sl]
        tb = base + c * CHG
        pltpu.sync_copy(g0_hbm.at[pl.ds(tb, CHG)], i0)
        pltpu.sync_copy(g1_hbm.at[pl.ds(tb, CHG)], i1)
        gin[sl] = (pltpu.async_copy(y_hbm.at[i0], r0, gs[0]),
                   pltpu.async_copy(y_hbm.at[i1], r1, gs[1]))

    issue_gather(0, 0)
    for c in range(nch):
        sl = c % 2
        if c + 1 < nch:
            nsl = (c + 1) % 2
            if wout[nsl] is not None:
                wout[nsl][0].wait()
                wout[nsl][1].wait()
            issue_gather(c + 1, nsl)
        r0, r1, _, _, _, ws = slots[sl]
        gin[sl][0].wait()
        gin[sl][1].wait()
        tb = base + c * CHG
        wout[sl] = (pltpu.async_copy(r0, y0_hbm.at[pl.ds(tb, CHG)], ws[0]),
                    pltpu.async_copy(r1, y1_hbm.at[pl.ds(tb, CHG)], ws[1]))
    last = (nch - 1) % 2
    for sl in (1 - last, last):
        if wout[sl] is not None:
            wout[sl][0].wait()
            wout[sl][1].wait()


def _gather(y, g0, g1):
    return pl.kernel(
        _gather_body,
        out_type=[jax.ShapeDtypeStruct((T, H), jnp.float32),
                  jax.ShapeDtypeStruct((T, H), jnp.float32)],
        mesh=_sc_mesh(),
        scratch_types=[pltpu.VMEM((CHG, H), jnp.float32),
                       pltpu.VMEM((CHG, H), jnp.float32),
                       pltpu.VMEM((CHG,), jnp.int32),
                       pltpu.VMEM((CHG,), jnp.int32),
                       pltpu.VMEM((CHG, H), jnp.float32),
                       pltpu.VMEM((CHG, H), jnp.float32),
                       pltpu.VMEM((CHG,), jnp.int32),
                       pltpu.VMEM((CHG,), jnp.int32)]
                     + [pltpu.SemaphoreType.DMA] * 8,
    )(y, g0, g1)


def _combine_body(y0_ref, y1_ref, w0_ref, w1_ref, out_ref):
    w0 = w0_ref[...]
    w1 = w1_ref[...]
    a = jnp.where(w0 > 0, y0_ref[...] * w0, 0.0)
    b = jnp.where(w1 > 0, y1_ref[...] * w1, 0.0)
    out_ref[...] = a + b


def _combine(y0, y1, w0, w1):
    col_i = pl.BlockSpec((TT, 1), lambda i: (i, 0))
    return pl.pallas_call(
        _combine_body,
        grid=(T // TT,),
        in_specs=[pl.BlockSpec((TT, H), lambda i: (i, 0)),
                  pl.BlockSpec((TT, H), lambda i: (i, 0)),
                  col_i, col_i],
        out_specs=pl.BlockSpec((TT, H), lambda i: (i, 0)),
        out_shape=jax.ShapeDtypeStruct((T, H), jnp.float32),
    )(y0, y1, w0, w1)


def _expert_stage(x, plan_outs, wg, wu, wd, mlp_bf16):
    _, s0, s1, g0, g1, w0, w1 = plan_outs
    buf = _dispatch(x, s0.reshape(T), s1.reshape(T))
    y = _mlp(buf, wg, wu, wd, mlp_bf16)
    y0, y1 = _gather(y, g0.reshape(T), g1.reshape(T))
    return y0, y1, w0, w1


def kernel(hidden_states, Wr0, Wg0, Wu0, Wd0, Wr1, Wg1, Wu1, Wd1):
    x = hidden_states.reshape(T, H)
    p0 = _plan(x, Wr0)
    y0, y1, w0, w1 = _expert_stage(x, p0, Wg0, Wu0, Wd0, False)
    x2, *p1 = _combine_plan(y0, y1, w0, w1, Wr1)
    y0, y1, w0, w1 = _expert_stage(x2, p1, Wg1, Wu1, Wd1, True)
    x3 = _combine(y0, y1, w0, w1)
    op = x3.reshape(hidden_states.shape)
    return op, jnp.concatenate([p0[0], p1[0]], axis=0)


# speedup vs baseline: 1.0715x; 1.0034x over previous
import functools

import jax
import jax.numpy as jnp
from jax import lax
from jax.experimental import pallas as pl
from jax.experimental.pallas import tpu as pltpu
from jax.experimental.pallas import tpu_sc as plsc

T, H, E, K, I, C = 2048, 2048, 8, 2, 5504, 512
TRASH = E * C
BUF_ROWS = E * C + C
TT = 256
IT = 128
NC, NS = 2, 16
NW = NC * NS
TOK_W = T // NW
CHUNK = 16


def _plan_math(x, wr_ref, logits_ref, s0_ref, s1_ref, g0_ref, g1_ref,
               w0_ref, w1_ref, base_ref):
    logits = jnp.dot(x, wr_ref[...], preferred_element_type=jnp.float32)
    logits_ref[...] = logits

    m = jnp.max(logits, axis=-1, keepdims=True)
    ex = jnp.exp(logits - m)
    p = ex / jnp.sum(ex, axis=-1, keepdims=True)

    iota_e = lax.broadcasted_iota(jnp.int32, (TT, E), 1)
    v0 = jnp.max(p, axis=-1, keepdims=True)
    i0 = jnp.min(jnp.where(p == v0, iota_e, E), axis=-1, keepdims=True)
    pm = jnp.where(iota_e == i0, -1.0, p)
    v1 = jnp.max(pm, axis=-1, keepdims=True)
    i1 = jnp.min(jnp.where(pm == v1, iota_e, E), axis=-1, keepdims=True)
    s = v0 + v1
    w0 = v0 / s
    w1 = v1 / s

    oh0 = (iota_e == i0).astype(jnp.float32)
    oh1 = (iota_e == i1).astype(jnp.float32)
    ohs = oh0 + oh1
    r = lax.broadcasted_iota(jnp.int32, (TT, TT), 0)
    c = lax.broadcasted_iota(jnp.int32, (TT, TT), 1)
    ltri = (r > c).astype(jnp.float32)
    cnt = base_ref[...] + jnp.dot(ltri, ohs, preferred_element_type=jnp.float32)
    pos0 = jnp.sum(cnt * oh0, axis=-1, keepdims=True)
    pos1 = jnp.sum(cnt * oh1, axis=-1, keepdims=True)
    keep0 = pos0 < C
    keep1 = pos1 < C
    slot0 = jnp.where(keep0, i0 * C + pos0.astype(jnp.int32), TRASH)
    slot1 = jnp.where(keep1, i1 * C + pos1.astype(jnp.int32), TRASH)

    s0_ref[...] = slot0
    s1_ref[...] = slot1
    g0_ref[...] = jnp.minimum(slot0, TRASH - 1)
    g1_ref[...] = jnp.minimum(slot1, TRASH - 1)
    w0_ref[...] = jnp.where(keep0, w0, 0.0)
    w1_ref[...] = jnp.where(keep1, w1, 0.0)
    base_ref[...] = base_ref[...] + jnp.sum(ohs, axis=0, keepdims=True)


def _plan_body(x_ref, wr_ref, logits_ref, s0_ref, s1_ref, g0_ref, g1_ref,
               w0_ref, w1_ref, base_ref):
    @pl.when(pl.program_id(0) == 0)
    def _():
        base_ref[...] = jnp.zeros_like(base_ref)

    _plan_math(x_ref[...], wr_ref, logits_ref, s0_ref, s1_ref, g0_ref,
               g1_ref, w0_ref, w1_ref, base_ref)


_PLAN_OUT_SHAPE = [jax.ShapeDtypeStruct((T, E), jnp.float32),
                   jax.ShapeDtypeStruct((T, 1), jnp.int32),
                   jax.ShapeDtypeStruct((T, 1), jnp.int32),
                   jax.ShapeDtypeStruct((T, 1), jnp.int32),
                   jax.ShapeDtypeStruct((T, 1), jnp.int32),
                   jax.ShapeDtypeStruct((T, 1), jnp.float32),
                   jax.ShapeDtypeStruct((T, 1), jnp.float32)]


def _plan(x, wr):
    col_i = pl.BlockSpec((TT, 1), lambda i: (i, 0))
    return pl.pallas_call(
        _plan_body,
        grid=(T // TT,),
        in_specs=[pl.BlockSpec((TT, H), lambda i: (i, 0)),
                  pl.BlockSpec((H, E), lambda i: (0, 0))],
        out_specs=[pl.BlockSpec((TT, E), lambda i: (i, 0)),
                   col_i, col_i, col_i, col_i, col_i, col_i],
        out_shape=list(_PLAN_OUT_SHAPE),
        scratch_shapes=[pltpu.VMEM((1, E), jnp.float32)],
    )(x, wr)


def _combine_plan_body(y0_ref, y1_ref, cw0_ref, cw1_ref, wr_ref, x_ref,
                       logits_ref, s0_ref, s1_ref, g0_ref, g1_ref,
                       w0_ref, w1_ref, base_ref):
    @pl.when(pl.program_id(0) == 0)
    def _():
        base_ref[...] = jnp.zeros_like(base_ref)

    cw0 = cw0_ref[...]
    cw1 = cw1_ref[...]
    x = (jnp.where(cw0 > 0, y0_ref[...] * cw0, 0.0)
         + jnp.where(cw1 > 0, y1_ref[...] * cw1, 0.0))
    x_ref[...] = x
    _plan_math(x, wr_ref, logits_ref, s0_ref, s1_ref, g0_ref, g1_ref,
               w0_ref, w1_ref, base_ref)


def _combine_plan(y0, y1, cw0, cw1, wr):
    col_i = pl.BlockSpec((TT, 1), lambda i: (i, 0))
    return pl.pallas_call(
        _combine_plan_body,
        grid=(T // TT,),
        in_specs=[pl.BlockSpec((TT, H), lambda i: (i, 0)),
                  pl.BlockSpec((TT, H), lambda i: (i, 0)),
                  col_i, col_i,
                  pl.BlockSpec((H, E), lambda i: (0, 0))],
        out_specs=[pl.BlockSpec((TT, H), lambda i: (i, 0)),
                   pl.BlockSpec((TT, E), lambda i: (i, 0)),
                   col_i, col_i, col_i, col_i, col_i, col_i],
        out_shape=[jax.ShapeDtypeStruct((T, H), jnp.float32)]
                  + list(_PLAN_OUT_SHAPE),
        scratch_shapes=[pltpu.VMEM((1, E), jnp.float32)],
    )(y0, y1, cw0, cw1, wr)


def _sc_mesh():
    return plsc.VectorSubcoreMesh(core_axis_name="c", subcore_axis_name="s",
                                  num_cores=NC)


def _dispatch_body(x_hbm, s0_hbm, s1_hbm, buf_hbm,
                   xr_a, i0_a, i1_a, xr_b, i0_b, i1_b,
                   ssem_a, ssem_b, w0_a, w1_a, w0_b, w1_b):
    wid = lax.axis_index("s") * NC + lax.axis_index("c")
    base = wid * TOK_W
    nch = TOK_W // CHUNK
    slots = ((xr_a, i0_a, i1_a, ssem_a, w0_a, w1_a),
             (xr_b, i0_b, i1_b, ssem_b, w0_b, w1_b))
    stage_cp = [None, None]
    scat_cp = [None, None]

    def stage(c, sl):
        xr, i0, i1, ssem, _, _ = slots[sl]
        tb = base + c * CHUNK
        pltpu.sync_copy(s0_hbm.at[pl.ds(tb, CHUNK)], i0)
        pltpu.sync_copy(s1_hbm.at[pl.ds(tb, CHUNK)], i1)
        stage_cp[sl] = pltpu.async_copy(x_hbm.at[pl.ds(tb, CHUNK)], xr, ssem)

    stage(0, 0)
    for c in range(nch):
        sl = c % 2
        xr, i0, i1, _, ws0, ws1 = slots[sl]
        stage_cp[sl].wait()
        scat_cp[sl] = (pltpu.async_copy(xr, buf_hbm.at[i0], ws0),
                       pltpu.async_copy(xr, buf_hbm.at[i1], ws1))
        if c + 1 < nch:
            nsl = (c + 1) % 2
            if scat_cp[nsl] is not None:
                scat_cp[nsl][0].wait()
                scat_cp[nsl][1].wait()
            stage(c + 1, nsl)
    last = (nch - 1) % 2
    for sl in (1 - last, last):
        if scat_cp[sl] is not None:
            scat_cp[sl][0].wait()
            scat_cp[sl][1].wait()


def _dispatch(x, s0, s1):
    return pl.kernel(
        _dispatch_body,
        out_type=jax.ShapeDtypeStruct((BUF_ROWS, H), jnp.float32),
        mesh=_sc_mesh(),
        scratch_types=[pltpu.VMEM((CHUNK, H), jnp.float32),
                       pltpu.VMEM((CHUNK,), jnp.int32),
                       pltpu.VMEM((CHUNK,), jnp.int32),
                       pltpu.VMEM((CHUNK, H), jnp.float32),
                       pltpu.VMEM((CHUNK,), jnp.int32),
                       pltpu.VMEM((CHUNK,), jnp.int32),
                       pltpu.SemaphoreType.DMA, pltpu.SemaphoreType.DMA,
                       pltpu.SemaphoreType.DMA, pltpu.SemaphoreType.DMA,
                       pltpu.SemaphoreType.DMA, pltpu.SemaphoreType.DMA],
    )(x, s0, s1)


NQUAD = 10


def _mlp_body(bf16, buf_ref, wgq_ref, wuq_ref, wdq_ref, wgs_ref, wus_ref,
              wds_ref, y_ref, xbf_ref):
    it = pl.program_id(1)
    cdt = jnp.bfloat16 if bf16 else jnp.float32

    @pl.when(it == 0)
    def _():
        if bf16:
            xbf_ref[...] = buf_ref[...].astype(jnp.bfloat16)

    x = xbf_ref[...] if bf16 else buf_ref[...]

    def glu(wg2, wu2, wd2, first):
        g = jnp.dot(x, wg2.astype(cdt), preferred_element_type=jnp.float32)
        u = jnp.dot(x, wu2.astype(cdt), preferred_element_type=jnp.float32)
        h = (g * lax.logistic(g) * u).astype(cdt)
        yt = jnp.dot(h, wd2.astype(cdt), preferred_element_type=jnp.float32)
        y_ref[...] = yt if first else y_ref[...] + yt

    @pl.when(it == 0)
    def _():
        glu(wgq_ref[0], wuq_ref[0], wdq_ref[0], True)

    @pl.when((it > 0) & (it < NQUAD))
    def _():
        glu(wgq_ref[0], wuq_ref[0], wdq_ref[0], False)

    @pl.when(it >= NQUAD)
    def _():
        glu(wgs_ref[0], wus_ref[0], wds_ref[0], False)


def _mlp(buf, wg, wu, wd, bf16):
    def smap_i(e, i):
        return (e, 0, jnp.minimum(jnp.maximum(i, NQUAD) + 4 * NQUAD - NQUAD,
                                  I // IT - 1))

    def smap_d(e, i):
        return (e, jnp.minimum(jnp.maximum(i, NQUAD) + 4 * NQUAD - NQUAD,
                               I // IT - 1), 0)

    return pl.pallas_call(
        functools.partial(_mlp_body, bf16),
        grid=(E, NQUAD + 3),
        in_specs=[
            pl.BlockSpec((C, H), lambda e, i: (e, 0)),
            pl.BlockSpec((1, H, 4 * IT),
                         lambda e, i: (e, 0, jnp.minimum(i, NQUAD - 1))),
            pl.BlockSpec((1, H, 4 * IT),
                         lambda e, i: (e, 0, jnp.minimum(i, NQUAD - 1))),
            pl.BlockSpec((1, 4 * IT, H),
                         lambda e, i: (e, jnp.minimum(i, NQUAD - 1), 0)),
            pl.BlockSpec((1, H, IT), smap_i),
            pl.BlockSpec((1, H, IT), smap_i),
            pl.BlockSpec((1, IT, H), smap_d),
        ],
        out_specs=pl.BlockSpec((C, H), lambda e, i: (e, 0)),
        out_shape=jax.ShapeDtypeStruct((E * C, H), jnp.float32),
        scratch_shapes=[pltpu.VMEM((C, H), jnp.bfloat16)],
    )(buf, wg, wu, wd, wg, wu, wd)


CHG = 8


def _gather_body(y_hbm, g0_hbm, g1_hbm, y0_hbm, y1_hbm,
                 r0_a, r1_a, i0_a, i1_a, r0_b, r1_b, i0_b, i1_b,
                 g0s_a, g1s_a, g0s_b, g1s_b, w0s_a, w1s_a, w0s_b, w1s_b):
    wid = lax.axis_index("s") * NC + lax.axis_index("c")
    base = wid * TOK_W
    nch = TOK_W // CHG
    slots = ((r0_a, r1_a, i0_a, i1_a, (g0s_a, g1s_a), (w0s_a, w1s_a)),
             (r0_b, r1_b, i0_b, i1_b, (g0s_b, g1s_b), (w0s_b, w1s_b)))
    gin = [None, None]
    wout = [None, None]

    def issue_gather(c, sl):
        r0, r1, i0, i1, gs, _ = slots[sl]
        tb = base + c * CHG
        pltpu.sync_copy(g0_hbm.at[pl.ds(tb, CHG)], i0)
        pltpu.sync_copy(g1_hbm.at[pl.ds(tb, CHG)], i1)
        gin[sl] = (pltpu.async_copy(y_hbm.at[i0], r0, gs[0]),
                   pltpu.async_copy(y_hbm.at[i1], r1, gs[1]))

    issue_gather(0, 0)
    for c in range(nch):
        sl = c % 2
        if c + 1 < nch:
            nsl = (c + 1) % 2
            if wout[nsl] is not None:
                wout[nsl][0].wait()
                wout[nsl][1].wait()
            issue_gather(c + 1, nsl)
        r0, r1, _, _, _, ws = slots[sl]
        gin[sl][0].wait()
        gin[sl][1].wait()
        tb = base + c * CHG
        wout[sl] = (pltpu.async_copy(r0, y0_hbm.at[pl.ds(tb, CHG)], ws[0]),
                    pltpu.async_copy(r1, y1_hbm.at[pl.ds(tb, CHG)], ws[1]))
    last = (nch - 1) % 2
    for sl in (1 - last, last):
        if wout[sl] is not None:
            wout[sl][0].wait()
            wout[sl][1].wait()


def _gather(y, g0, g1):
    return pl.kernel(
        _gather_body,
        out_type=[jax.ShapeDtypeStruct((T, H), jnp.float32),
                  jax.ShapeDtypeStruct((T, H), jnp.float32)],
        mesh=_sc_mesh(),
        scratch_types=[pltpu.VMEM((CHG, H), jnp.float32),
                       pltpu.VMEM((CHG, H), jnp.float32),
                       pltpu.VMEM((CHG,), jnp.int32),
                       pltpu.VMEM((CHG,), jnp.int32),
                       pltpu.VMEM((CHG, H), jnp.float32),
                       pltpu.VMEM((CHG, H), jnp.float32),
                       pltpu.VMEM((CHG,), jnp.int32),
                       pltpu.VMEM((CHG,), jnp.int32)]
                     + [pltpu.SemaphoreType.DMA] * 8,
    )(y, g0, g1)


def _combine_body(y0_ref, y1_ref, w0_ref, w1_ref, out_ref):
    w0 = w0_ref[...]
    w1 = w1_ref[...]
    a = jnp.where(w0 > 0, y0_ref[...] * w0, 0.0)
    b = jnp.where(w1 > 0, y1_ref[...] * w1, 0.0)
    out_ref[...] = a + b


def _combine(y0, y1, w0, w1):
    col_i = pl.BlockSpec((TT, 1), lambda i: (i, 0))
    return pl.pallas_call(
        _combine_body,
        grid=(T // TT,),
        in_specs=[pl.BlockSpec((TT, H), lambda i: (i, 0)),
                  pl.BlockSpec((TT, H), lambda i: (i, 0)),
                  col_i, col_i],
        out_specs=pl.BlockSpec((TT, H), lambda i: (i, 0)),
        out_shape=jax.ShapeDtypeStruct((T, H), jnp.float32),
    )(y0, y1, w0, w1)


def _expert_stage(x, plan_outs, wg, wu, wd, mlp_bf16):
    _, s0, s1, g0, g1, w0, w1 = plan_outs
    buf = _dispatch(x, s0.reshape(T), s1.reshape(T))
    y = _mlp(buf, wg, wu, wd, mlp_bf16)
    y0, y1 = _gather(y, g0.reshape(T), g1.reshape(T))
    return y0, y1, w0, w1


def kernel(hidden_states, Wr0, Wg0, Wu0, Wd0, Wr1, Wg1, Wu1, Wd1):
    x = hidden_states.reshape(T, H)
    p0 = _plan(x, Wr0)
    y0, y1, w0, w1 = _expert_stage(x, p0, Wg0, Wu0, Wd0, False)
    x2, *p1 = _combine_plan(y0, y1, w0, w1, Wr1)
    y0, y1, w0, w1 = _expert_stage(x2, p1, Wg1, Wu1, Wd1, True)
    x3 = _combine(y0, y1, w0, w1)
    op = x3.reshape(hidden_states.shape)
    return op, jnp.concatenate([p0[0], p1[0]], axis=0)
